# baseline (speedup 1.0000x reference)
import functools

import jax
import jax.numpy as jnp
from jax import lax
from jax.experimental import pallas as pl
from jax.experimental.pallas import tpu as pltpu
from jax.experimental.pallas import tpu_sc as plsc

N_NODES = 10000
N_EDGES = 320000
FEATS = 128

NC = 2
NS = 16
NW = NC * NS
N_PAD = 10240
PAD_TILE = N_PAD // NS
Z_TILE = 640
Z_TILE_LAST = N_NODES - 15 * Z_TILE

CHUNK = 80
E_PER_TILE = N_EDGES // NW
AGG_CHUNKS = E_PER_TILE // CHUNK
DEG_E_PER_TILE = N_EDGES // NS
DEG_CHUNKS = DEG_E_PER_TILE // CHUNK

_MESH = plsc.VectorSubcoreMesh(core_axis_name="c", subcore_axis_name="s")


def _deg_body(src_hbm, dst_hbm, out_hbm, deg_sh, zbuf, ones_v, iv, *sems):
    c = lax.axis_index("c")
    s = lax.axis_index("s")
    for j in range(PAD_TILE // 16):
        zbuf[pl.ds(j * 16, 16)] = jnp.zeros((16,), jnp.float32)
    pltpu.sync_copy(zbuf, deg_sh.at[pl.ds(s * PAD_TILE, PAD_TILE)])
    for j in range(5):
        ones_v[pl.ds(j * 16, 16)] = jnp.ones((16,), jnp.float32)

    @pl.when(c == 0)
    def _():
        pltpu.sync_copy(src_hbm.at[pl.ds(s * DEG_E_PER_TILE, DEG_E_PER_TILE)],
                        iv)

    @pl.when(c == 1)
    def _():
        pltpu.sync_copy(dst_hbm.at[pl.ds(s * DEG_E_PER_TILE, DEG_E_PER_TILE)],
                        iv)

    plsc.subcore_barrier()
    ones = ones_v.at[pl.ds(0, CHUNK)]

    def idx(j):
        return iv.at[pl.ds(j * CHUNK, CHUNK)]

    def drain(sem):
        pltpu.make_async_copy(ones, deg_sh.at[idx(0)], sem).wait()

    nsem = len(sems)

    def body(p, carry):
        for q in range(nsem):
            j = nsem * p + q

            @pl.when(j >= nsem)
            def _():
                drain(sems[q])

            pltpu.async_copy(ones, deg_sh.at[idx(j)], sems[q], add=True)
        return carry

    lax.fori_loop(0, DEG_CHUNKS // nsem, body, 0)
    for q in range(DEG_CHUNKS % nsem):
        j = (DEG_CHUNKS // nsem) * nsem + q
        drain(sems[q])
        pltpu.async_copy(ones, deg_sh.at[idx(j)], sems[q], add=True)
    for q in range(nsem):
        drain(sems[q])
    plsc.subcore_barrier()
    pltpu.sync_copy(deg_sh.at[pl.ds(s * PAD_TILE, PAD_TILE)],
                    out_hbm.at[pl.ds(c * N_PAD + s * PAD_TILE, PAD_TILE)])


_deg_call = functools.partial(
    pl.kernel,
    out_type=jax.ShapeDtypeStruct((NC * N_PAD,), jnp.float32),
    mesh=_MESH,
    scratch_types=[
        pltpu.VMEM_SHARED((N_PAD,), jnp.float32),
        pltpu.VMEM((PAD_TILE,), jnp.float32),
        pltpu.VMEM((80,), jnp.float32),
        pltpu.VMEM((DEG_E_PER_TILE,), jnp.int32),
    ] + [pltpu.SemaphoreType.DMA for _ in range(8)],
)(_deg_body)


def _agg_body(xs_hbm, src_hbm, dst_hbm, out_hbm, z_sh, sv, dv, rows0, rows1,
              rows2, gsem0, gsem1, gsem2):
    c = lax.axis_index("c")
    s = lax.axis_index("s")
    w = c * NS + s

    for j in range(CHUNK):
        for l in range(8):
            rows0[j, pl.ds(l * 16, 16)] = jnp.zeros((16,), jnp.float32)
    nz = lax.select(s == 15, Z_TILE_LAST // CHUNK, Z_TILE // CHUNK)

    def zbody(m, carry):
        pltpu.sync_copy(rows0, z_sh.at[pl.ds(s * Z_TILE + m * CHUNK, CHUNK)])
        return carry

    lax.fori_loop(0, nz, zbody, 0)

    pltpu.sync_copy(src_hbm.at[pl.ds(w * E_PER_TILE, E_PER_TILE)], sv)
    pltpu.sync_copy(dst_hbm.at[pl.ds(w * E_PER_TILE, E_PER_TILE)], dv)
    plsc.subcore_barrier()

    def sidx(j):
        return sv.at[pl.ds(j * CHUNK, CHUNK)]

    def didx(j):
        return dv.at[pl.ds(j * CHUNK, CHUNK)]

    bufs = (rows0, rows1, rows2)
    sems = (gsem0, gsem1, gsem2)
    for q in range(3):
        pltpu.async_copy(xs_hbm.at[sidx(q)], bufs[q], sems[q])

    def body(p, carry):
        for q in range(3):
            j = 3 * p + q
            pltpu.make_async_copy(xs_hbm.at[sidx(0)], bufs[q],
                                  sems[q]).wait()
            pltpu.sync_copy(bufs[q], z_sh.at[didx(j)], add=True)

            @pl.when(j + 3 < AGG_CHUNKS)
            def _():
                pltpu.async_copy(xs_hbm.at[sidx(j + 3)], bufs[q], sems[q])

        return carry

    lax.fori_loop(0, AGG_CHUNKS // 3, body, 0)
    for q in range(AGG_CHUNKS % 3):
        j = (AGG_CHUNKS // 3) * 3 + q
        pltpu.make_async_copy(xs_hbm.at[sidx(0)], bufs[q], sems[q]).wait()
        pltpu.sync_copy(bufs[q], z_sh.at[didx(j)], add=True)
    plsc.subcore_barrier()

    @pl.when(s < 15)
    def _():
        pltpu.sync_copy(z_sh.at[pl.ds(s * Z_TILE, Z_TILE)],
                        out_hbm.at[pl.ds(c * N_PAD + s * Z_TILE, Z_TILE)])

    @pl.when(s == 15)
    def _():
        pltpu.sync_copy(
            z_sh.at[pl.ds(15 * Z_TILE, Z_TILE_LAST)],
            out_hbm.at[pl.ds(c * N_PAD + 15 * Z_TILE, Z_TILE_LAST)])


_agg_call = functools.partial(
    pl.kernel,
    out_type=jax.ShapeDtypeStruct((NC * N_PAD, FEATS), jnp.float32),
    mesh=_MESH,
    scratch_types=[
        pltpu.VMEM_SHARED((N_NODES, FEATS), jnp.float32),
        pltpu.VMEM((E_PER_TILE,), jnp.int32),
        pltpu.VMEM((E_PER_TILE,), jnp.int32),
        pltpu.VMEM((CHUNK, FEATS), jnp.float32),
        pltpu.VMEM((CHUNK, FEATS), jnp.float32),
        pltpu.VMEM((CHUNK, FEATS), jnp.float32),
        pltpu.SemaphoreType.DMA,
        pltpu.SemaphoreType.DMA,
        pltpu.SemaphoreType.DMA,
    ],
)(_agg_body)


ROWS_BLK = 1000


def _mm_body(x_ref, w_ref, o_ref):
    o_ref[...] = jnp.dot(x_ref[...], w_ref[...],
                         preferred_element_type=jnp.float32)


_mm_call = pl.pallas_call(
    _mm_body,
    grid=(N_NODES // ROWS_BLK,),
    in_specs=[
        pl.BlockSpec((ROWS_BLK, FEATS), lambda i: (i, 0)),
        pl.BlockSpec((FEATS, FEATS), lambda i: (0, 0)),
    ],
    out_specs=pl.BlockSpec((ROWS_BLK, FEATS), lambda i: (i, 0)),
    out_shape=jax.ShapeDtypeStruct((N_NODES, FEATS), jnp.float32),
)


def _scale_body(x_ref, dg_ref, o_ref):
    d = dg_ref[...]
    norm = jnp.where(d > 0, lax.rsqrt(jnp.maximum(d, 1.0)), 0.0)
    o_ref[...] = x_ref[...] * norm


_scale_call = pl.pallas_call(
    _scale_body,
    grid=(N_NODES // ROWS_BLK,),
    in_specs=[
        pl.BlockSpec((ROWS_BLK, FEATS), lambda i: (i, 0)),
        pl.BlockSpec((ROWS_BLK, 1), lambda i: (i, 0)),
    ],
    out_specs=pl.BlockSpec((ROWS_BLK, FEATS), lambda i: (i, 0)),
    out_shape=jax.ShapeDtypeStruct((N_NODES, FEATS), jnp.float32),
)


PAD_BLK = 1024
N_BLOCKS = N_PAD // PAD_BLK


def _readout_body(z0_ref, z1_ref, dg_ref, b_ref, w1_ref, b1_ref,
                  w2_ref, b2_ref, out_ref, acc_ref):
    i = pl.program_id(0)

    @pl.when(i == 0)
    def _():
        acc_ref[...] = jnp.zeros_like(acc_ref)

    z = z0_ref[...] + z1_ref[...]
    d = dg_ref[...]
    norm = jnp.where(d > 0, lax.rsqrt(jnp.maximum(d, 1.0)), 0.0)
    h = jnp.maximum(z * norm + b_ref[...], 0.0)
    row = i * PAD_BLK + lax.broadcasted_iota(jnp.int32, (PAD_BLK, 1), 0)
    h = jnp.where(row < N_NODES, h, 0.0)
    acc_ref[...] += jnp.sum(h, axis=0, keepdims=True)

    @pl.when(i == pl.num_programs(0) - 1)
    def _():
        mean = acc_ref[...] * (1.0 / N_NODES)
        o1 = jnp.dot(mean, w1_ref[...], preferred_element_type=jnp.float32)
        o1 = jnp.maximum(o1 + b1_ref[...], 0.0)
        out_ref[...] = (
            jnp.dot(o1, w2_ref[...], preferred_element_type=jnp.float32)
            + b2_ref[...])


_readout_call = pl.pallas_call(
    _readout_body,
    grid=(N_BLOCKS,),
    in_specs=[
        pl.BlockSpec((PAD_BLK, FEATS), lambda i: (i, 0)),
        pl.BlockSpec((PAD_BLK, FEATS), lambda i: (i + N_BLOCKS, 0)),
        pl.BlockSpec((PAD_BLK, 1), lambda i: (i + N_BLOCKS, 0)),
        pl.BlockSpec((1, FEATS), lambda i: (0, 0)),
        pl.BlockSpec((FEATS, FEATS), lambda i: (0, 0)),
        pl.BlockSpec((1, FEATS), lambda i: (0, 0)),
        pl.BlockSpec((FEATS, 10), lambda i: (0, 0)),
        pl.BlockSpec((1, 10), lambda i: (0, 0)),
    ],
    out_specs=pl.BlockSpec((1, 10), lambda i: (0, 0)),
    out_shape=jax.ShapeDtypeStruct((1, 10), jnp.float32),
    scratch_shapes=[pltpu.VMEM((1, 128), jnp.float32)],
)


def kernel(in_feat, edge_index, W, b, W1, b1, W2, b2):
    edges = edge_index.astype(jnp.int32)
    src = edges[0]
    dst = edges[1]
    y = _mm_call(in_feat, W)
    degs = _deg_call(src, dst)
    degs2d = degs.reshape(2 * N_PAD, 1)
    xs = _scale_call(y, degs2d)
    zp = _agg_call(xs, src, dst)
    return _readout_call(
        zp, zp, degs2d, b.reshape(1, FEATS), W1, b1.reshape(1, FEATS),
        W2, b2.reshape(1, 10))


# speedup vs baseline: 1.0800x; 1.0800x over previous
import functools

import jax
import jax.numpy as jnp
from jax import lax
from jax.experimental import pallas as pl
from jax.experimental.pallas import tpu as pltpu
from jax.experimental.pallas import tpu_sc as plsc

N_NODES = 10000
N_EDGES = 320000
FEATS = 128

NC = 2
NS = 16
NW = NC * NS
N_PAD = 10240
PAD_TILE = N_PAD // NS
Z_TILE = 640
Z_TILE_LAST = N_NODES - 15 * Z_TILE

CHUNK = 80
E_PER_TILE = N_EDGES // NW
AGG_CHUNKS = E_PER_TILE // CHUNK
DEG_E_PER_TILE = N_EDGES // NS
DEG_CHUNKS = DEG_E_PER_TILE // CHUNK

_MESH = plsc.VectorSubcoreMesh(core_axis_name="c", subcore_axis_name="s")


def _deg_body(src_hbm, dst_hbm, out_hbm, deg_sh, zbuf, ones_v, iv, *sems):
    c = lax.axis_index("c")
    s = lax.axis_index("s")
    for j in range(PAD_TILE // 16):
        zbuf[pl.ds(j * 16, 16)] = jnp.zeros((16,), jnp.float32)
    pltpu.sync_copy(zbuf, deg_sh.at[pl.ds(s * PAD_TILE, PAD_TILE)])
    for j in range(5):
        ones_v[pl.ds(j * 16, 16)] = jnp.ones((16,), jnp.float32)

    @pl.when(c == 0)
    def _():
        pltpu.sync_copy(src_hbm.at[pl.ds(s * DEG_E_PER_TILE, DEG_E_PER_TILE)],
                        iv)

    @pl.when(c == 1)
    def _():
        pltpu.sync_copy(dst_hbm.at[pl.ds(s * DEG_E_PER_TILE, DEG_E_PER_TILE)],
                        iv)

    plsc.subcore_barrier()
    ones = ones_v.at[pl.ds(0, CHUNK)]

    def idx(j):
        return iv.at[pl.ds(j * CHUNK, CHUNK)]

    def drain(sem):
        pltpu.make_async_copy(ones, deg_sh.at[idx(0)], sem).wait()

    nsem = len(sems)

    def body(p, carry):
        for q in range(nsem):
            j = nsem * p + q

            @pl.when(j >= nsem)
            def _():
                drain(sems[q])

            pltpu.async_copy(ones, deg_sh.at[idx(j)], sems[q], add=True)
        return carry

    lax.fori_loop(0, DEG_CHUNKS // nsem, body, 0)
    for q in range(DEG_CHUNKS % nsem):
        j = (DEG_CHUNKS // nsem) * nsem + q
        drain(sems[q])
        pltpu.async_copy(ones, deg_sh.at[idx(j)], sems[q], add=True)
    for q in range(nsem):
        drain(sems[q])
    plsc.subcore_barrier()
    pltpu.sync_copy(deg_sh.at[pl.ds(s * PAD_TILE, PAD_TILE)],
                    out_hbm.at[pl.ds(c * N_PAD + s * PAD_TILE, PAD_TILE)])


_deg_call = functools.partial(
    pl.kernel,
    out_type=jax.ShapeDtypeStruct((NC * N_PAD,), jnp.float32),
    mesh=_MESH,
    scratch_types=[
        pltpu.VMEM_SHARED((N_PAD,), jnp.float32),
        pltpu.VMEM((PAD_TILE,), jnp.float32),
        pltpu.VMEM((80,), jnp.float32),
        pltpu.VMEM((DEG_E_PER_TILE,), jnp.int32),
    ] + [pltpu.SemaphoreType.DMA for _ in range(8)],
)(_deg_body)


def _agg_body(xs_hbm, src_hbm, dst_hbm, out_hbm, z_sh, sv, dv, rows0, rows1,
              rows2, gsem0, gsem1, gsem2):
    c = lax.axis_index("c")
    s = lax.axis_index("s")
    w = c * NS + s

    for j in range(CHUNK):
        for l in range(8):
            rows0[j, pl.ds(l * 16, 16)] = jnp.zeros((16,), jnp.float32)
    nz = lax.select(s == 15, Z_TILE_LAST // CHUNK, Z_TILE // CHUNK)

    def zbody(m, carry):
        pltpu.sync_copy(rows0, z_sh.at[pl.ds(s * Z_TILE + m * CHUNK, CHUNK)])
        return carry

    lax.fori_loop(0, nz, zbody, 0)

    pltpu.sync_copy(src_hbm.at[pl.ds(w * E_PER_TILE, E_PER_TILE)], sv)
    pltpu.sync_copy(dst_hbm.at[pl.ds(w * E_PER_TILE, E_PER_TILE)], dv)
    plsc.subcore_barrier()

    def sidx(j):
        return sv.at[pl.ds(j * CHUNK, CHUNK)]

    def didx(j):
        return dv.at[pl.ds(j * CHUNK, CHUNK)]

    bufs = (rows0, rows1, rows2)
    sems = (gsem0, gsem1, gsem2)
    for q in range(3):
        pltpu.async_copy(xs_hbm.at[sidx(q)], bufs[q], sems[q])

    def body(p, carry):
        for q in range(3):
            j = 3 * p + q
            pltpu.make_async_copy(xs_hbm.at[sidx(0)], bufs[q],
                                  sems[q]).wait()
            pltpu.sync_copy(bufs[q], z_sh.at[didx(j)], add=True)

            @pl.when(j + 3 < AGG_CHUNKS)
            def _():
                pltpu.async_copy(xs_hbm.at[sidx(j + 3)], bufs[q], sems[q])

        return carry

    lax.fori_loop(0, AGG_CHUNKS // 3, body, 0)
    for q in range(AGG_CHUNKS % 3):
        j = (AGG_CHUNKS // 3) * 3 + q
        pltpu.make_async_copy(xs_hbm.at[sidx(0)], bufs[q], sems[q]).wait()
        pltpu.sync_copy(bufs[q], z_sh.at[didx(j)], add=True)
    plsc.subcore_barrier()

    @pl.when(s < 15)
    def _():
        pltpu.sync_copy(z_sh.at[pl.ds(s * Z_TILE, Z_TILE)],
                        out_hbm.at[pl.ds(c * N_PAD + s * Z_TILE, Z_TILE)])

    @pl.when(s == 15)
    def _():
        pltpu.sync_copy(
            z_sh.at[pl.ds(15 * Z_TILE, Z_TILE_LAST)],
            out_hbm.at[pl.ds(c * N_PAD + 15 * Z_TILE, Z_TILE_LAST)])


_agg_call = functools.partial(
    pl.kernel,
    out_type=jax.ShapeDtypeStruct((NC * N_PAD, FEATS), jnp.float32),
    mesh=_MESH,
    scratch_types=[
        pltpu.VMEM_SHARED((N_NODES, FEATS), jnp.float32),
        pltpu.VMEM((E_PER_TILE,), jnp.int32),
        pltpu.VMEM((E_PER_TILE,), jnp.int32),
        pltpu.VMEM((CHUNK, FEATS), jnp.float32),
        pltpu.VMEM((CHUNK, FEATS), jnp.float32),
        pltpu.VMEM((CHUNK, FEATS), jnp.float32),
        pltpu.SemaphoreType.DMA,
        pltpu.SemaphoreType.DMA,
        pltpu.SemaphoreType.DMA,
    ],
)(_agg_body)


def _split_body(e_ref, s_ref, d_ref):
    s_ref[...] = e_ref[0, :]
    d_ref[...] = e_ref[1, :]


_split_call = pl.pallas_call(
    _split_body,
    out_shape=[
        jax.ShapeDtypeStruct((N_EDGES,), jnp.int32),
        jax.ShapeDtypeStruct((N_EDGES,), jnp.int32),
    ],
)


ROWS_BLK = 1000


def _mm_body(x_ref, w_ref, o_ref):
    o_ref[...] = jnp.dot(x_ref[...], w_ref[...],
                         preferred_element_type=jnp.float32)


_mm_call = pl.pallas_call(
    _mm_body,
    grid=(N_NODES // ROWS_BLK,),
    in_specs=[
        pl.BlockSpec((ROWS_BLK, FEATS), lambda i: (i, 0)),
        pl.BlockSpec((FEATS, FEATS), lambda i: (0, 0)),
    ],
    out_specs=pl.BlockSpec((ROWS_BLK, FEATS), lambda i: (i, 0)),
    out_shape=jax.ShapeDtypeStruct((N_NODES, FEATS), jnp.float32),
)


def _scale_body(x_ref, dg_ref, o_ref):
    d = dg_ref[...]
    norm = jnp.where(d > 0, lax.rsqrt(jnp.maximum(d, 1.0)), 0.0)
    o_ref[...] = x_ref[...] * norm


_scale_call = pl.pallas_call(
    _scale_body,
    grid=(N_NODES // ROWS_BLK,),
    in_specs=[
        pl.BlockSpec((ROWS_BLK, FEATS), lambda i: (i, 0)),
        pl.BlockSpec((ROWS_BLK, 1), lambda i: (i, 0)),
    ],
    out_specs=pl.BlockSpec((ROWS_BLK, FEATS), lambda i: (i, 0)),
    out_shape=jax.ShapeDtypeStruct((N_NODES, FEATS), jnp.float32),
)


PAD_BLK = 1024
N_BLOCKS = N_PAD // PAD_BLK


def _readout_body(z0_ref, z1_ref, dg_ref, b_ref, w1_ref, b1_ref,
                  w2_ref, b2_ref, out_ref, acc_ref):
    i = pl.program_id(0)

    @pl.when(i == 0)
    def _():
        acc_ref[...] = jnp.zeros_like(acc_ref)

    z = z0_ref[...] + z1_ref[...]
    d = dg_ref[...]
    norm = jnp.where(d > 0, lax.rsqrt(jnp.maximum(d, 1.0)), 0.0)
    h = jnp.maximum(z * norm + b_ref[...], 0.0)
    row = i * PAD_BLK + lax.broadcasted_iota(jnp.int32, (PAD_BLK, 1), 0)
    h = jnp.where(row < N_NODES, h, 0.0)
    acc_ref[...] += jnp.sum(h, axis=0, keepdims=True)

    @pl.when(i == pl.num_programs(0) - 1)
    def _():
        mean = acc_ref[...] * (1.0 / N_NODES)
        o1 = jnp.dot(mean, w1_ref[...], preferred_element_type=jnp.float32)
        o1 = jnp.maximum(o1 + b1_ref[...], 0.0)
        out_ref[...] = (
            jnp.dot(o1, w2_ref[...], preferred_element_type=jnp.float32)
            + b2_ref[...])


_readout_call = pl.pallas_call(
    _readout_body,
    grid=(N_BLOCKS,),
    in_specs=[
        pl.BlockSpec((PAD_BLK, FEATS), lambda i: (i, 0)),
        pl.BlockSpec((PAD_BLK, FEATS), lambda i: (i + N_BLOCKS, 0)),
        pl.BlockSpec((PAD_BLK, 1), lambda i: (i + N_BLOCKS, 0)),
        pl.BlockSpec((1, FEATS), lambda i: (0, 0)),
        pl.BlockSpec((FEATS, FEATS), lambda i: (0, 0)),
        pl.BlockSpec((1, FEATS), lambda i: (0, 0)),
        pl.BlockSpec((FEATS, 10), lambda i: (0, 0)),
        pl.BlockSpec((1, 10), lambda i: (0, 0)),
    ],
    out_specs=pl.BlockSpec((1, 10), lambda i: (0, 0)),
    out_shape=jax.ShapeDtypeStruct((1, 10), jnp.float32),
    scratch_shapes=[pltpu.VMEM((1, 128), jnp.float32)],
)


def kernel(in_feat, edge_index, W, b, W1, b1, W2, b2):
    edges = edge_index.astype(jnp.int32)
    src, dst = _split_call(edges)
    y = _mm_call(in_feat, W)
    degs = _deg_call(src, dst)
    degs2d = degs.reshape(2 * N_PAD, 1)
    xs = _scale_call(y, degs2d)
    zp = _agg_call(xs, src, dst)
    return _readout_call(
        zp, zp, degs2d, b.reshape(1, FEATS), W1, b1.reshape(1, FEATS),
        W2, b2.reshape(1, 10))


# speedup vs baseline: 1.1318x; 1.0480x over previous
import functools

import jax
import jax.numpy as jnp
from jax import lax
from jax.experimental import pallas as pl
from jax.experimental.pallas import tpu as pltpu
from jax.experimental.pallas import tpu_sc as plsc

N_NODES = 10000
N_EDGES = 320000
FEATS = 128

NC = 2
NS = 16
NW = NC * NS
N_PAD = 10240
PAD_TILE = N_PAD // NS
Z_TILE = 640
Z_TILE_LAST = N_NODES - 15 * Z_TILE

CHUNK = 80
E_PER_TILE = N_EDGES // NW
AGG_CHUNKS = E_PER_TILE // CHUNK
DEG_E_PER_TILE = N_EDGES // NS
DEG_CHUNKS = DEG_E_PER_TILE // CHUNK

_MESH = plsc.VectorSubcoreMesh(core_axis_name="c", subcore_axis_name="s")


def _deg_body(src_hbm, dst_hbm, out_hbm, deg_sh, zbuf, ones_v, iv, *sems):
    c = lax.axis_index("c")
    s = lax.axis_index("s")
    for j in range(PAD_TILE // 16):
        zbuf[pl.ds(j * 16, 16)] = jnp.zeros((16,), jnp.float32)
    pltpu.sync_copy(zbuf, deg_sh.at[pl.ds(s * PAD_TILE, PAD_TILE)])
    for j in range(5):
        ones_v[pl.ds(j * 16, 16)] = jnp.ones((16,), jnp.float32)

    @pl.when(c == 0)
    def _():
        pltpu.sync_copy(src_hbm.at[pl.ds(s * DEG_E_PER_TILE, DEG_E_PER_TILE)],
                        iv)

    @pl.when(c == 1)
    def _():
        pltpu.sync_copy(dst_hbm.at[pl.ds(s * DEG_E_PER_TILE, DEG_E_PER_TILE)],
                        iv)

    plsc.subcore_barrier()
    ones = ones_v.at[pl.ds(0, CHUNK)]

    def idx(j):
        return iv.at[pl.ds(j * CHUNK, CHUNK)]

    def drain(sem):
        pltpu.make_async_copy(ones, deg_sh.at[idx(0)], sem).wait()

    nsem = len(sems)

    def body(p, carry):
        for q in range(nsem):
            j = nsem * p + q

            @pl.when(j >= nsem)
            def _():
                drain(sems[q])

            pltpu.async_copy(ones, deg_sh.at[idx(j)], sems[q], add=True)
        return carry

    lax.fori_loop(0, DEG_CHUNKS // nsem, body, 0)
    for q in range(DEG_CHUNKS % nsem):
        j = (DEG_CHUNKS // nsem) * nsem + q
        drain(sems[q])
        pltpu.async_copy(ones, deg_sh.at[idx(j)], sems[q], add=True)
    for q in range(nsem):
        drain(sems[q])
    plsc.subcore_barrier()
    pltpu.sync_copy(deg_sh.at[pl.ds(s * PAD_TILE, PAD_TILE)],
                    out_hbm.at[pl.ds(c * N_PAD + s * PAD_TILE, PAD_TILE)])


_deg_call = functools.partial(
    pl.kernel,
    out_type=jax.ShapeDtypeStruct((NC * N_PAD,), jnp.float32),
    mesh=_MESH,
    scratch_types=[
        pltpu.VMEM_SHARED((N_PAD,), jnp.float32),
        pltpu.VMEM((PAD_TILE,), jnp.float32),
        pltpu.VMEM((80,), jnp.float32),
        pltpu.VMEM((DEG_E_PER_TILE,), jnp.int32),
    ] + [pltpu.SemaphoreType.DMA for _ in range(8)],
)(_deg_body)


def _agg_body(xs_hbm, src_hbm, dst_hbm, out_hbm, z_sh, sv, dv, rows0, rows1,
              rows2, gsem0, gsem1, gsem2):
    c = lax.axis_index("c")
    s = lax.axis_index("s")
    w = c * NS + s

    for j in range(CHUNK):
        for l in range(8):
            rows0[j, pl.ds(l * 16, 16)] = jnp.zeros((16,), jnp.float32)
    nz = lax.select(s == 15, Z_TILE_LAST // CHUNK, Z_TILE // CHUNK)

    def zbody(m, carry):
        pltpu.sync_copy(rows0, z_sh.at[pl.ds(s * Z_TILE + m * CHUNK, CHUNK)])
        return carry

    lax.fori_loop(0, nz, zbody, 0)

    pltpu.sync_copy(src_hbm.at[pl.ds(w * E_PER_TILE, E_PER_TILE)], sv)
    pltpu.sync_copy(dst_hbm.at[pl.ds(w * E_PER_TILE, E_PER_TILE)], dv)
    plsc.subcore_barrier()

    def sidx(j):
        return sv.at[pl.ds(j * CHUNK, CHUNK)]

    def didx(j):
        return dv.at[pl.ds(j * CHUNK, CHUNK)]

    bufs = (rows0, rows1, rows2)
    sems = (gsem0, gsem1, gsem2)
    for q in range(3):
        pltpu.async_copy(xs_hbm.at[sidx(q)], bufs[q], sems[q])

    def body(p, carry):
        for q in range(3):
            j = 3 * p + q
            pltpu.make_async_copy(xs_hbm.at[sidx(0)], bufs[q],
                                  sems[q]).wait()
            pltpu.sync_copy(bufs[q], z_sh.at[didx(j)], add=True)

            @pl.when(j + 3 < AGG_CHUNKS)
            def _():
                pltpu.async_copy(xs_hbm.at[sidx(j + 3)], bufs[q], sems[q])

        return carry

    lax.fori_loop(0, AGG_CHUNKS // 3, body, 0)
    for q in range(AGG_CHUNKS % 3):
        j = (AGG_CHUNKS // 3) * 3 + q
        pltpu.make_async_copy(xs_hbm.at[sidx(0)], bufs[q], sems[q]).wait()
        pltpu.sync_copy(bufs[q], z_sh.at[didx(j)], add=True)
    plsc.subcore_barrier()

    @pl.when(s < 15)
    def _():
        pltpu.sync_copy(z_sh.at[pl.ds(s * Z_TILE, Z_TILE)],
                        out_hbm.at[pl.ds(c * N_PAD + s * Z_TILE, Z_TILE)])

    @pl.when(s == 15)
    def _():
        pltpu.sync_copy(
            z_sh.at[pl.ds(15 * Z_TILE, Z_TILE_LAST)],
            out_hbm.at[pl.ds(c * N_PAD + 15 * Z_TILE, Z_TILE_LAST)])


_agg_call = functools.partial(
    pl.kernel,
    out_type=jax.ShapeDtypeStruct((NC * N_PAD, FEATS), jnp.float32),
    mesh=_MESH,
    scratch_types=[
        pltpu.VMEM_SHARED((N_NODES, FEATS), jnp.float32),
        pltpu.VMEM((E_PER_TILE,), jnp.int32),
        pltpu.VMEM((E_PER_TILE,), jnp.int32),
        pltpu.VMEM((CHUNK, FEATS), jnp.float32),
        pltpu.VMEM((CHUNK, FEATS), jnp.float32),
        pltpu.VMEM((CHUNK, FEATS), jnp.float32),
        pltpu.SemaphoreType.DMA,
        pltpu.SemaphoreType.DMA,
        pltpu.SemaphoreType.DMA,
    ],
)(_agg_body)


def _split_body(e_ref, s_ref, d_ref):
    s_ref[...] = e_ref[0, :]
    d_ref[...] = e_ref[1, :]


_split_call = pl.pallas_call(
    _split_body,
    out_shape=[
        jax.ShapeDtypeStruct((N_EDGES,), jnp.int32),
        jax.ShapeDtypeStruct((N_EDGES,), jnp.int32),
    ],
)


PAD_BLK = 1024
DEG_BLK = PAD_BLK // 128


def _mm_body(x_ref, w_ref, o_ref):
    o_ref[...] = jnp.dot(x_ref[...], w_ref[...],
                         preferred_element_type=jnp.float32)


_mm_call = pl.pallas_call(
    _mm_body,
    grid=(N_PAD // PAD_BLK,),
    in_specs=[
        pl.BlockSpec((PAD_BLK, FEATS), lambda i: (i, 0)),
        pl.BlockSpec((FEATS, FEATS), lambda i: (0, 0)),
    ],
    out_specs=pl.BlockSpec((PAD_BLK, FEATS), lambda i: (i, 0)),
    out_shape=jax.ShapeDtypeStruct((N_PAD, FEATS), jnp.float32),
)


def _expand_col(v8):
    r8 = lax.broadcasted_iota(jnp.int32, (PAD_BLK, 8), 0) // 128
    c8 = lax.broadcasted_iota(jnp.int32, (PAD_BLK, 8), 1)
    sel = (r8 == c8).astype(jnp.float32)
    tmp = jnp.dot(sel, v8, preferred_element_type=jnp.float32)
    lane = lax.broadcasted_iota(jnp.int32, (PAD_BLK, 128), 1)
    row = lax.broadcasted_iota(jnp.int32, (PAD_BLK, 128), 0) % 128
    return jnp.sum(jnp.where(lane == row, tmp, 0.0), axis=1, keepdims=True)


def _scale_body(x_ref, dg_ref, o_ref):
    d8 = dg_ref[...]
    n8 = jnp.where(d8 > 0, lax.rsqrt(jnp.maximum(d8, 1.0)), 0.0)
    o_ref[...] = x_ref[...] * _expand_col(n8)


_scale_call = pl.pallas_call(
    _scale_body,
    grid=(N_PAD // PAD_BLK,),
    in_specs=[
        pl.BlockSpec((PAD_BLK, FEATS), lambda i: (i, 0)),
        pl.BlockSpec((DEG_BLK, 128), lambda i: (i, 0)),
    ],
    out_specs=pl.BlockSpec((PAD_BLK, FEATS), lambda i: (i, 0)),
    out_shape=jax.ShapeDtypeStruct((N_PAD, FEATS), jnp.float32),
)


N_BLOCKS = N_PAD // PAD_BLK


def _readout_body(z0_ref, z1_ref, dg_ref, b_ref, w1_ref, b1_ref,
                  w2_ref, b2_ref, out_ref, acc_ref):
    i = pl.program_id(0)

    @pl.when(i == 0)
    def _():
        acc_ref[...] = jnp.zeros_like(acc_ref)

    z = z0_ref[...] + z1_ref[...]
    d8 = dg_ref[...]
    n8 = jnp.where(d8 > 0, lax.rsqrt(jnp.maximum(d8, 1.0)), 0.0)
    h = jnp.maximum(z * _expand_col(n8) + b_ref[...], 0.0)
    row = i * PAD_BLK + lax.broadcasted_iota(jnp.int32, (PAD_BLK, 1), 0)
    h = jnp.where(row < N_NODES, h, 0.0)
    acc_ref[...] += jnp.sum(h, axis=0, keepdims=True)

    @pl.when(i == pl.num_programs(0) - 1)
    def _():
        mean = acc_ref[...] * (1.0 / N_NODES)
        o1 = jnp.dot(mean, w1_ref[...], preferred_element_type=jnp.float32)
        o1 = jnp.maximum(o1 + b1_ref[...], 0.0)
        out_ref[...] = (
            jnp.dot(o1, w2_ref[...], preferred_element_type=jnp.float32)
            + b2_ref[...])


_readout_call = pl.pallas_call(
    _readout_body,
    grid=(N_BLOCKS,),
    in_specs=[
        pl.BlockSpec((PAD_BLK, FEATS), lambda i: (i, 0)),
        pl.BlockSpec((PAD_BLK, FEATS), lambda i: (i + N_BLOCKS, 0)),
        pl.BlockSpec((DEG_BLK, 128), lambda i: (i + N_BLOCKS, 0)),
        pl.BlockSpec((1, FEATS), lambda i: (0, 0)),
        pl.BlockSpec((FEATS, FEATS), lambda i: (0, 0)),
        pl.BlockSpec((1, FEATS), lambda i: (0, 0)),
        pl.BlockSpec((FEATS, 10), lambda i: (0, 0)),
        pl.BlockSpec((1, 10), lambda i: (0, 0)),
    ],
    out_specs=pl.BlockSpec((1, 10), lambda i: (0, 0)),
    out_shape=jax.ShapeDtypeStruct((1, 10), jnp.float32),
    scratch_shapes=[pltpu.VMEM((1, 128), jnp.float32)],
)


def kernel(in_feat, edge_index, W, b, W1, b1, W2, b2):
    edges = edge_index.astype(jnp.int32)
    src, dst = _split_call(edges)
    y = _mm_call(in_feat, W)
    degs = _deg_call(src, dst)
    degs_mat = degs.reshape(2 * N_PAD // 128, 128)
    xs = _scale_call(y, degs_mat)
    zp = _agg_call(xs, src, dst)
    return _readout_call(
        zp, zp, degs_mat, b.reshape(1, FEATS), W1, b1.reshape(1, FEATS),
        W2, b2.reshape(1, 10))
